# baseline (speedup 1.0000x reference)
import functools

import jax
import jax.numpy as jnp
from jax import lax
from jax.experimental import pallas as pl
from jax.experimental.pallas import tpu as pltpu
from jax.experimental.pallas import tpu_sc as plsc

N_NODES = 10000
N_PAD = 10112
D = 128
NUM_GRAPHS = 64
E_EDGES = 320000
NUM_WORKERS = 32
CHUNK = 128
CHUNKS = 80
NBUF = 2
BLK = 8
DEG_FAN = 4
SB_C0 = 1
SB_C1 = 9
SB_CAP = max(SB_C0, SB_C1)
CAP_CHUNKS = SB_CAP * 16
NBLKS = CAP_CHUNKS // BLK
E_CORE0 = 16 * SB_C0 * 16 * CHUNK
E_PAD = NUM_WORKERS * CHUNKS * CHUNK
ROWS_PER_TILE = N_PAD // 16

_HIGH = jax.lax.Precision.HIGHEST


def _sc_mesh():
    return plsc.VectorSubcoreMesh(core_axis_name="c", subcore_axis_name="s")


@functools.partial(
    pl.kernel,
    out_type=jax.ShapeDtypeStruct((2, N_PAD, D), jnp.float32),
    mesh=_sc_mesh(),
    scratch_types=[
        pltpu.VMEM((CHUNKS, CHUNK), jnp.int32),
        pltpu.VMEM((CHUNK, D), jnp.float32),
        pltpu.VMEM_SHARED((N_PAD, D), jnp.float32),
        pltpu.SemaphoreType.DMA,
        pltpu.SemaphoreType.DMA,
        pltpu.SemaphoreType.DMA,
        pltpu.SemaphoreType.DMA,
    ],
)
def _deg_sc(dst_hbm, ones_hbm, zeros_hbm, out_hbm, dst_v, ones_v, acc,
            s0, s1, s2, s3):
    sems = [s0, s1, s2, s3]
    c = lax.axis_index("c")
    s = lax.axis_index("s")
    w = c * 16 + s
    pltpu.sync_copy(dst_hbm.at[w], dst_v)
    pltpu.sync_copy(ones_hbm, ones_v)
    sl = pl.ds(s * ROWS_PER_TILE, ROWS_PER_TILE)
    pltpu.sync_copy(zeros_hbm.at[sl], acc.at[sl])
    plsc.subcore_barrier()

    def body(g, carry):
        for b in range(DEG_FAN):
            pltpu.async_copy(ones_v, acc.at[dst_v.at[g * DEG_FAN + b]],
                             sems[b], add=True)
        for b in range(DEG_FAN):
            pltpu.make_async_copy(ones_v, acc.at[dst_v.at[0]], sems[b]).wait()
        return carry

    lax.fori_loop(0, CHUNKS // DEG_FAN, body, 0)
    plsc.subcore_barrier()
    pltpu.sync_copy(acc.at[sl], out_hbm.at[c, sl])


@functools.partial(
    pl.kernel,
    out_type=jax.ShapeDtypeStruct((2, N_PAD, D), jnp.float32),
    mesh=_sc_mesh(),
    scratch_types=[
        pltpu.VMEM((BLK, CHUNK), jnp.int32),
        pltpu.VMEM((BLK, CHUNK), jnp.int32),
        pltpu.VMEM((BLK, CHUNK), jnp.int32),
        pltpu.VMEM((BLK, CHUNK), jnp.int32),
        pltpu.VMEM((CHUNK, D), jnp.float32),
        pltpu.VMEM_SHARED((N_PAD, D), jnp.float32),
        pltpu.SemaphoreType.DMA,
        pltpu.SemaphoreType.DMA,
        pltpu.SemaphoreType.DMA,
    ],
)
def _scatter_sc(h_hbm, src_hbm, dst_hbm, out_hbm,
                si0, si1, di0, di1, rbuf, acc, gsem, i0, i1):
    sidx = [si0, si1]
    didx = [di0, di1]
    isem = [i0, i1]
    c = lax.axis_index("c")
    s = lax.axis_index("s")
    w = c * 16 + s

    def fill_idx(slot, blk):
        pltpu.async_copy(src_hbm.at[w, blk], sidx[slot], isem[slot])
        pltpu.async_copy(dst_hbm.at[w, blk], didx[slot], isem[slot])

    def wait_idx(slot):
        pltpu.make_async_copy(src_hbm.at[w, 0], sidx[slot], isem[slot]).wait()
        pltpu.make_async_copy(dst_hbm.at[w, 0], didx[slot], isem[slot]).wait()

    gmax = jnp.where(c == 0, SB_C0, SB_C1)

    pltpu.sync_copy(src_hbm.at[w, 0], sidx[0])
    pltpu.sync_copy(dst_hbm.at[w, 0], didx[0])
    fill_idx(1, 1)
    sl = pl.ds(s * ROWS_PER_TILE, ROWS_PER_TILE)
    pltpu.sync_copy(h_hbm.at[sl], acc.at[sl])
    plsc.subcore_barrier()

    def body(G, carry):
        for t in range(16):
            if t == 0:
                @pl.when(G >= 1)
                def _():
                    wait_idx(0)
                    fill_idx(1, 2 * G + 1)
            elif t == 8:
                wait_idx(1)

                @pl.when(G < gmax - 1)
                def _():
                    fill_idx(0, 2 * G + 2)
            p, r = t // 8, t % 8
            pltpu.async_copy(h_hbm.at[sidx[p].at[r]], rbuf, gsem).wait()
            pltpu.sync_copy(rbuf, acc.at[didx[p].at[r]], add=True)
        return carry

    lax.fori_loop(0, gmax, body, 0)
    plsc.subcore_barrier()
    pltpu.sync_copy(acc.at[sl], out_hbm.at[c, sl])


def _row_mask():
    rows = lax.broadcasted_iota(jnp.int32, (N_PAD, 1), 0)
    return (rows < N_NODES).astype(jnp.float32)


def _prep_body(x_ref, w_ref, deg_ref, h_ref, dinv_ref):
    dinv = lax.rsqrt(jnp.maximum(deg_ref[0] + deg_ref[1] + 1.0, 1.0))
    h = jnp.dot(x_ref[...], w_ref[...], precision=_HIGH,
                preferred_element_type=jnp.float32)
    h_ref[...] = h * dinv * _row_mask()
    dinv_ref[...] = dinv


def _mid_body(acc_ref, hp_ref, dinv_ref, b_ref, g_ref, be_ref, w_ref, out_ref):
    dinv = dinv_ref[...]
    z = (acc_ref[0] + acc_ref[1] - hp_ref[...]) * dinv + b_ref[...]
    zq = z[:N_NODES]
    m = jnp.mean(zq, axis=0, keepdims=True)
    v = jnp.mean((zq - m) ** 2, axis=0, keepdims=True)
    a = jnp.maximum((zq - m) * lax.rsqrt(v + 1e-5) * g_ref[...] + be_ref[...],
                    0.0)
    h = jnp.dot(a, w_ref[...], precision=_HIGH,
                preferred_element_type=jnp.float32) * dinv[:N_NODES]
    out_ref[pl.ds(0, N_NODES), :] = h
    out_ref[pl.ds(N_NODES, N_PAD - N_NODES), :] = jnp.zeros(
        (N_PAD - N_NODES, D), jnp.float32)


def _final_body(acc_ref, hp_ref, dinv_ref, b_ref, batch_ref, out_ref):
    dinv = dinv_ref[...]
    z = (acc_ref[0] + acc_ref[1] - hp_ref[...]) * dinv + b_ref[...]
    zq = z[:N_NODES]
    gid = lax.broadcasted_iota(jnp.int32, (N_NODES, NUM_GRAPHS), 1)
    onehot = (batch_ref[...] == gid).astype(jnp.float32)
    out_ref[...] = lax.dot_general(
        onehot, zq, (((0,), (0,)), ((), ())), precision=_HIGH,
        preferred_element_type=jnp.float32)


_prep_tc = pl.pallas_call(
    _prep_body, out_shape=[jax.ShapeDtypeStruct((N_PAD, D), jnp.float32),
                           jax.ShapeDtypeStruct((N_PAD, D), jnp.float32)])

_mid_tc = pl.pallas_call(
    _mid_body, out_shape=jax.ShapeDtypeStruct((N_PAD, D), jnp.float32))

_final_tc = pl.pallas_call(
    _final_body, out_shape=jax.ShapeDtypeStruct((NUM_GRAPHS, D), jnp.float32))


def kernel(x, edge_index, batch, W1, b1, g1, be1, W2, b2, g2, be2, W3, b3):
    pad = E_PAD - E_EDGES
    fill = jnp.full((pad,), N_NODES, jnp.int32)
    dst3 = jnp.concatenate([edge_index[1], fill]).reshape(NUM_WORKERS, CHUNKS, CHUNK)

    cap = CAP_CHUNKS * CHUNK

    def pack(ids):
        e0 = ids[:E_CORE0].reshape(16, SB_C0 * 16 * CHUNK)
        e0 = jnp.concatenate(
            [e0, jnp.full((16, cap - SB_C0 * 16 * CHUNK), N_NODES, jnp.int32)],
            axis=1)
        n1 = E_EDGES - E_CORE0
        e1 = jnp.concatenate(
            [ids[E_CORE0:], jnp.full((16 * cap - n1,), N_NODES, jnp.int32)])
        e1 = e1.reshape(16, cap)
        return jnp.concatenate([e0, e1], axis=0).reshape(
            NUM_WORKERS, NBLKS, BLK, CHUNK)

    src4 = pack(edge_index[0])
    dst4 = pack(edge_index[1])
    xp = jnp.concatenate(
        [x, jnp.zeros((N_PAD - N_NODES, D), jnp.float32)], axis=0)
    ones_wide = jnp.ones((CHUNK, D), jnp.float32)
    zeros_wide = jnp.zeros((N_PAD, D), jnp.float32)

    deg = _deg_sc(dst3, ones_wide, zeros_wide)
    h1, dinv = _prep_tc(xp, W1, deg)
    acc1 = _scatter_sc(h1, src4, dst4)
    h2 = _mid_tc(acc1, h1, dinv, b1.reshape(1, D), g1.reshape(1, D),
                 be1.reshape(1, D), W2)
    acc2 = _scatter_sc(h2, src4, dst4)
    h3 = _mid_tc(acc2, h2, dinv, b2.reshape(1, D), g2.reshape(1, D),
                 be2.reshape(1, D), W3)
    acc3 = _scatter_sc(h3, src4, dst4)
    return _final_tc(acc3, h3, dinv, b3.reshape(1, D),
                     batch.reshape(N_NODES, 1))


# speedup vs baseline: 1.2445x; 1.2445x over previous
import functools

import jax
import jax.numpy as jnp
from jax import lax
from jax.experimental import pallas as pl
from jax.experimental.pallas import tpu as pltpu
from jax.experimental.pallas import tpu_sc as plsc

N_NODES = 10000
N_PAD = 10112
D = 128
NUM_GRAPHS = 64
E_EDGES = 320000
NUM_WORKERS = 32
CHUNK = 128
CHUNKS = 80
NBUF = 2
BLK = 8
DEG_FAN = 4
SB_C0 = 5
SB_C1 = 5
SB_CAP = max(SB_C0, SB_C1)
CAP_CHUNKS = SB_CAP * 16
NBLKS = CAP_CHUNKS // BLK
E_CORE0 = 16 * SB_C0 * 16 * CHUNK
E_PAD = NUM_WORKERS * CHUNKS * CHUNK
ROWS_PER_TILE = N_PAD // 16

_HIGH = jax.lax.Precision.HIGHEST


def _sc_mesh():
    return plsc.VectorSubcoreMesh(core_axis_name="c", subcore_axis_name="s")


@functools.partial(
    pl.kernel,
    out_type=jax.ShapeDtypeStruct((2, N_PAD, D), jnp.float32),
    mesh=_sc_mesh(),
    scratch_types=[
        pltpu.VMEM((CHUNKS, CHUNK), jnp.int32),
        pltpu.VMEM((CHUNK, D), jnp.float32),
        pltpu.VMEM_SHARED((N_PAD, D), jnp.float32),
        pltpu.SemaphoreType.DMA,
        pltpu.SemaphoreType.DMA,
        pltpu.SemaphoreType.DMA,
        pltpu.SemaphoreType.DMA,
    ],
)
def _deg_sc(dst_hbm, ones_hbm, zeros_hbm, out_hbm, dst_v, ones_v, acc,
            s0, s1, s2, s3):
    sems = [s0, s1, s2, s3]
    c = lax.axis_index("c")
    s = lax.axis_index("s")
    w = c * 16 + s
    pltpu.sync_copy(dst_hbm.at[w], dst_v)
    pltpu.sync_copy(ones_hbm, ones_v)
    sl = pl.ds(s * ROWS_PER_TILE, ROWS_PER_TILE)
    pltpu.sync_copy(zeros_hbm.at[sl], acc.at[sl])
    plsc.subcore_barrier()

    def body(g, carry):
        for b in range(DEG_FAN):
            pltpu.async_copy(ones_v, acc.at[dst_v.at[g * DEG_FAN + b]],
                             sems[b], add=True)
        for b in range(DEG_FAN):
            pltpu.make_async_copy(ones_v, acc.at[dst_v.at[0]], sems[b]).wait()
        return carry

    lax.fori_loop(0, CHUNKS // DEG_FAN, body, 0)
    plsc.subcore_barrier()
    pltpu.sync_copy(acc.at[sl], out_hbm.at[c, sl])


@functools.partial(
    pl.kernel,
    out_type=jax.ShapeDtypeStruct((2, N_PAD, D), jnp.float32),
    mesh=_sc_mesh(),
    scratch_types=[
        pltpu.VMEM((BLK, CHUNK), jnp.int32),
        pltpu.VMEM((BLK, CHUNK), jnp.int32),
        pltpu.VMEM((BLK, CHUNK), jnp.int32),
        pltpu.VMEM((BLK, CHUNK), jnp.int32),
        pltpu.VMEM((CHUNK, D), jnp.float32),
        pltpu.VMEM((CHUNK, D), jnp.float32),
        pltpu.VMEM_SHARED((N_PAD, D), jnp.float32),
        pltpu.SemaphoreType.DMA,
        pltpu.SemaphoreType.DMA,
        pltpu.SemaphoreType.DMA,
    ],
)
def _scatter_sc(h_hbm, src_hbm, dst_hbm, out_hbm,
                si0, si1, di0, di1, r0, r1, acc, gsem, i0, i1):
    sidx = [si0, si1]
    didx = [di0, di1]
    bufs = [r0, r1]
    isem = [i0, i1]
    c = lax.axis_index("c")
    s = lax.axis_index("s")
    w = c * 16 + s

    def fill_idx(slot, blk):
        pltpu.async_copy(src_hbm.at[w, blk], sidx[slot], isem[slot])
        pltpu.async_copy(dst_hbm.at[w, blk], didx[slot], isem[slot])

    def wait_idx(slot):
        pltpu.make_async_copy(src_hbm.at[w, 0], sidx[slot], isem[slot]).wait()
        pltpu.make_async_copy(dst_hbm.at[w, 0], didx[slot], isem[slot]).wait()

    def wait_gather(b):
        pltpu.make_async_copy(h_hbm.at[sidx[0].at[0]], bufs[b],
                              gsem).wait()

    pltpu.sync_copy(src_hbm.at[w, 0], sidx[0])
    pltpu.sync_copy(dst_hbm.at[w, 0], didx[0])
    fill_idx(1, 1)
    pltpu.async_copy(h_hbm.at[sidx[0].at[0]], bufs[0], gsem)
    sl = pl.ds(s * ROWS_PER_TILE, ROWS_PER_TILE)
    pltpu.sync_copy(h_hbm.at[sl], acc.at[sl])
    plsc.subcore_barrier()

    SB = SB_C0

    def body(G, carry):
        for t in range(16):
            b = t % 2
            bb = (b + 1) % 2
            if t == 0:
                @pl.when(G >= 1)
                def _():
                    fill_idx(1, 2 * G + 1)
            elif t == 8:
                @pl.when(G < SB - 1)
                def _():
                    fill_idx(0, 2 * G + 2)
            wait_gather(b)
            if t == 7:
                wait_idx(1)
                pltpu.async_copy(h_hbm.at[sidx[1].at[0]], bufs[bb], gsem)
            elif t == 15:
                @pl.when(G < SB - 1)
                def _():
                    wait_idx(0)
                    pltpu.async_copy(h_hbm.at[sidx[0].at[0]], bufs[bb], gsem)
            else:
                pg, rg = (t + 1) // 8, (t + 1) % 8
                pltpu.async_copy(h_hbm.at[sidx[pg].at[rg]], bufs[bb], gsem)
            pltpu.sync_copy(bufs[b], acc.at[didx[t // 8].at[t % 8]], add=True)
        return carry

    lax.fori_loop(0, SB, body, 0)
    plsc.subcore_barrier()
    pltpu.sync_copy(acc.at[sl], out_hbm.at[c, sl])


def _row_mask():
    rows = lax.broadcasted_iota(jnp.int32, (N_PAD, 1), 0)
    return (rows < N_NODES).astype(jnp.float32)


def _prep_body(x_ref, w_ref, deg_ref, h_ref, dinv_ref):
    dinv = lax.rsqrt(jnp.maximum(deg_ref[0] + deg_ref[1] + 1.0, 1.0))
    h = jnp.dot(x_ref[...], w_ref[...], precision=_HIGH,
                preferred_element_type=jnp.float32)
    h_ref[...] = h * dinv * _row_mask()
    dinv_ref[...] = dinv


def _mid_body(acc_ref, hp_ref, dinv_ref, b_ref, g_ref, be_ref, w_ref, out_ref):
    dinv = dinv_ref[...]
    z = (acc_ref[0] + acc_ref[1] - hp_ref[...]) * dinv + b_ref[...]
    zq = z[:N_NODES]
    m = jnp.mean(zq, axis=0, keepdims=True)
    v = jnp.mean((zq - m) ** 2, axis=0, keepdims=True)
    a = jnp.maximum((zq - m) * lax.rsqrt(v + 1e-5) * g_ref[...] + be_ref[...],
                    0.0)
    h = jnp.dot(a, w_ref[...], precision=_HIGH,
                preferred_element_type=jnp.float32) * dinv[:N_NODES]
    out_ref[pl.ds(0, N_NODES), :] = h
    out_ref[pl.ds(N_NODES, N_PAD - N_NODES), :] = jnp.zeros(
        (N_PAD - N_NODES, D), jnp.float32)


def _final_body(acc_ref, hp_ref, dinv_ref, b_ref, batch_ref, out_ref):
    dinv = dinv_ref[...]
    z = (acc_ref[0] + acc_ref[1] - hp_ref[...]) * dinv + b_ref[...]
    zq = z[:N_NODES]
    gid = lax.broadcasted_iota(jnp.int32, (N_NODES, NUM_GRAPHS), 1)
    onehot = (batch_ref[...] == gid).astype(jnp.float32)
    out_ref[...] = lax.dot_general(
        onehot, zq, (((0,), (0,)), ((), ())), precision=_HIGH,
        preferred_element_type=jnp.float32)


_prep_tc = pl.pallas_call(
    _prep_body, out_shape=[jax.ShapeDtypeStruct((N_PAD, D), jnp.float32),
                           jax.ShapeDtypeStruct((N_PAD, D), jnp.float32)])

_mid_tc = pl.pallas_call(
    _mid_body, out_shape=jax.ShapeDtypeStruct((N_PAD, D), jnp.float32))

_final_tc = pl.pallas_call(
    _final_body, out_shape=jax.ShapeDtypeStruct((NUM_GRAPHS, D), jnp.float32))


def kernel(x, edge_index, batch, W1, b1, g1, be1, W2, b2, g2, be2, W3, b3):
    pad = E_PAD - E_EDGES
    fill = jnp.full((pad,), N_NODES, jnp.int32)
    dst3 = jnp.concatenate([edge_index[1], fill]).reshape(NUM_WORKERS, CHUNKS, CHUNK)

    cap = CAP_CHUNKS * CHUNK

    def pack(ids):
        e0 = ids[:E_CORE0].reshape(16, SB_C0 * 16 * CHUNK)
        e0 = jnp.concatenate(
            [e0, jnp.full((16, cap - SB_C0 * 16 * CHUNK), N_NODES, jnp.int32)],
            axis=1)
        n1 = E_EDGES - E_CORE0
        e1 = jnp.concatenate(
            [ids[E_CORE0:], jnp.full((16 * cap - n1,), N_NODES, jnp.int32)])
        e1 = e1.reshape(16, cap)
        return jnp.concatenate([e0, e1], axis=0).reshape(
            NUM_WORKERS, NBLKS, BLK, CHUNK)

    src4 = pack(edge_index[0])
    dst4 = pack(edge_index[1])
    xp = jnp.concatenate(
        [x, jnp.zeros((N_PAD - N_NODES, D), jnp.float32)], axis=0)
    ones_wide = jnp.ones((CHUNK, D), jnp.float32)
    zeros_wide = jnp.zeros((N_PAD, D), jnp.float32)

    deg = _deg_sc(dst3, ones_wide, zeros_wide)
    h1, dinv = _prep_tc(xp, W1, deg)
    acc1 = _scatter_sc(h1, src4, dst4)
    h2 = _mid_tc(acc1, h1, dinv, b1.reshape(1, D), g1.reshape(1, D),
                 be1.reshape(1, D), W2)
    acc2 = _scatter_sc(h2, src4, dst4)
    h3 = _mid_tc(acc2, h2, dinv, b2.reshape(1, D), g2.reshape(1, D),
                 be2.reshape(1, D), W3)
    acc3 = _scatter_sc(h3, src4, dst4)
    return _final_tc(acc3, h3, dinv, b3.reshape(1, D),
                     batch.reshape(N_NODES, 1))
